# Initial kernel scaffold; baseline (speedup 1.0000x reference)
import jax
import jax.numpy as jnp
from jax.experimental import pallas as pl


def kernel(node_features, edge_indices, edge_weights, emb_table, W1, b1, W2, b2, W3, b3):
    raise NotImplementedError("write your pallas kernel here")


# speedup vs baseline: 4.3106x; 4.3106x over previous
import functools

import jax
import jax.numpy as jnp
from jax import lax
from jax.experimental import pallas as pl
from jax.experimental.pallas import tpu as pltpu
from jax.experimental.pallas import tpu_sc as plsc

N = 10000
NP = 10240
D = 256
HD = 128
V = 64000
T = 2
E = 160000
EP = 163840
CH = 128
NCHUNK = EP // 16 // CH
NGRP = 5
ROWS_PER_TILE = NP // 16

_MESH = plsc.VectorSubcoreMesh(core_axis_name="c", subcore_axis_name="s")


@functools.partial(
    pl.kernel,
    out_type=(
        jax.ShapeDtypeStruct((T, NP, D), jnp.float32),
        jax.ShapeDtypeStruct((T * NP,), jnp.float32),
    ),
    mesh=_MESH,
    scratch_types=(
        pltpu.VMEM((320,), jnp.int32),
        pltpu.VMEM((64, D), jnp.float32),
        pltpu.VMEM((16, CH), jnp.int32),
        pltpu.VMEM((16, CH), jnp.float32),
        pltpu.VMEM((ROWS_PER_TILE,), jnp.float32),
        pltpu.VMEM_SHARED((NP,), jnp.float32),
        pltpu.SemaphoreType.DMA,
    ),
)
def _sc_prep(emb_hbm, nf_hbm, dst_hbm, ew_hbm, x_out, deg_out,
             idx_v, rows_v, dst_buf, ew_buf, bounce, deg_acc, sem):
    c = lax.axis_index("c")
    s = lax.axis_index("s")
    wid = c * 16 + s

    for t in range(T):
        pltpu.sync_copy(nf_hbm.at[pl.ds(t * NP + wid * 320, 320)], idx_v)
        for j in range(5):
            pltpu.async_copy(emb_hbm.at[idx_v.at[pl.ds(j * 64, 64)]], rows_v,
                             sem).wait()
            pltpu.sync_copy(rows_v, x_out.at[t, pl.ds(wid * 320 + j * 64, 64)])

    def zero_body(i, _):
        bounce[pl.ds(i * 16, 16)] = jnp.zeros((16,), jnp.float32)
        return _
    lax.fori_loop(0, ROWS_PER_TILE // 16, zero_body, None)
    pltpu.sync_copy(bounce, deg_acc.at[pl.ds(s * ROWS_PER_TILE, ROWS_PER_TILE)])
    plsc.subcore_barrier()

    for h in range(2):
        @pl.when(c == h)
        def _():
            def grp_body(g, _):
                base = s * NCHUNK + g * 16
                pltpu.sync_copy(dst_hbm.at[h, pl.ds(base, 16)], dst_buf)
                pltpu.sync_copy(ew_hbm.at[h, pl.ds(base, 16)], ew_buf)
                for j in range(16):
                    pltpu.sync_copy(ew_buf.at[j], deg_acc.at[dst_buf.at[j]],
                                    add=True)
                return _
            lax.fori_loop(0, NGRP, grp_body, None)

    plsc.subcore_barrier()
    pltpu.sync_copy(deg_acc.at[pl.ds(s * ROWS_PER_TILE, ROWS_PER_TILE)], bounce)
    for h in range(2):
        @pl.when(c == h)
        def _():
            pltpu.sync_copy(
                bounce,
                deg_out.at[pl.ds(h * NP + s * ROWS_PER_TILE, ROWS_PER_TILE)])


@functools.partial(
    pl.kernel,
    out_type=jax.ShapeDtypeStruct((T, 2, NP, HD), jnp.float32),
    mesh=_MESH,
    scratch_types=(
        pltpu.VMEM((16, CH), jnp.int32),
        pltpu.VMEM((16, CH), jnp.int32),
        pltpu.VMEM((16, CH), jnp.float32),
        pltpu.VMEM((CH, HD), jnp.float32),
        pltpu.VMEM_SHARED((NP, HD), jnp.float32),
        pltpu.SemaphoreType.DMA,
    ),
)
def _sc_msg(ytil_hbm, src_hbm, dst_hbm, ew_hbm, g_out,
            src_buf, dst_buf, ew_buf, rows_v, acc, sem):
    c = lax.axis_index("c")
    s = lax.axis_index("s")

    for h in range(2):
        @pl.when(c == h)
        def _():
            for t in range(T):
                def zrow(r, _):
                    for gg in range(8):
                        rows_v[r, pl.ds(gg * 16, 16)] = jnp.zeros((16,),
                                                                  jnp.float32)
                    return _
                lax.fori_loop(0, CH, zrow, None)
                for q in range(ROWS_PER_TILE // CH):
                    pltpu.sync_copy(
                        rows_v,
                        acc.at[pl.ds(s * ROWS_PER_TILE + q * CH, CH)])
                plsc.subcore_barrier()

                def grp_body(g, _):
                    base = s * NCHUNK + g * 16
                    pltpu.sync_copy(src_hbm.at[t, pl.ds(base, 16)], src_buf)
                    pltpu.sync_copy(dst_hbm.at[t, pl.ds(base, 16)], dst_buf)
                    pltpu.sync_copy(ew_hbm.at[t, pl.ds(base, 16)], ew_buf)

                    def chunk_body(j, _):
                        pltpu.async_copy(
                            ytil_hbm.at[t, h].at[src_buf.at[j]], rows_v,
                            sem).wait()

                        def scale(k, _):
                            ewv = ew_buf.at[j][pl.ds(k * 16, 16)]
                            for l in range(16):
                                w16 = jnp.broadcast_to(ewv[l], (16,))
                                r = k * 16 + l
                                for gg in range(8):
                                    sl = pl.ds(gg * 16, 16)
                                    rows_v[r, sl] = rows_v[r, sl] * w16
                            return _
                        lax.fori_loop(0, CH // 16, scale, None)
                        pltpu.sync_copy(rows_v, acc.at[dst_buf.at[j]],
                                        add=True)
                        return _
                    lax.fori_loop(0, 16, chunk_body, None)
                    return _
                lax.fori_loop(0, NGRP, grp_body, None)
                plsc.subcore_barrier()

                for q in range(ROWS_PER_TILE // CH):
                    base = s * ROWS_PER_TILE + q * CH
                    pltpu.sync_copy(acc.at[pl.ds(base, CH)], rows_v)
                    pltpu.sync_copy(rows_v, g_out.at[t, h, pl.ds(base, CH)])
                plsc.subcore_barrier()


RB = 512
NB = NP // RB
RBF = 400
NBF = N // RBF


def _tc1_body(x_ref, deg_ref, w_ref, y_ref):
    dis = lax.rsqrt(deg_ref[0] + 1.0)
    z = jnp.dot(x_ref[0], w_ref[...], preferred_element_type=jnp.float32)
    y = z * dis
    y_ref[0, 0] = y[:, :HD]
    y_ref[0, 1] = y[:, HD:]


def _tc_mid_body(g_ref, yt_ref, deg_ref, b_ref, w_ref, y_ref):
    dis = lax.rsqrt(deg_ref[0] + 1.0)
    a0 = (g_ref[0, 0] + yt_ref[0, 0]) * dis
    a1 = (g_ref[0, 1] + yt_ref[0, 1]) * dis
    hcat = jnp.concatenate([a0, a1], axis=1) + b_ref[...][None, :]
    z = jnp.dot(hcat, w_ref[...], preferred_element_type=jnp.float32)
    y = z * dis
    y_ref[0, 0] = y[:, :HD]
    y_ref[0, 1] = y[:, HD:]


def _tc_final_body(g_ref, yt_ref, deg_ref, b_ref, o_ref):
    dis = lax.rsqrt(deg_ref[0] + 1.0)
    o0 = (g_ref[0, 0] + yt_ref[0, 0]) * dis
    o1 = (g_ref[0, 1] + yt_ref[0, 1]) * dis
    o_ref[0] = jnp.concatenate([o0, o1], axis=1) + b_ref[...][None, :]


def _tc1(x, deg3, w):
    return pl.pallas_call(
        _tc1_body,
        grid=(T, NB),
        in_specs=[
            pl.BlockSpec((1, RB, D), lambda t, n: (t, n, 0)),
            pl.BlockSpec((1, RB, 1), lambda t, n: (t, n, 0)),
            pl.BlockSpec((D, D), lambda t, n: (0, 0)),
        ],
        out_specs=pl.BlockSpec((1, 2, RB, HD), lambda t, n: (t, 0, n, 0)),
        out_shape=jax.ShapeDtypeStruct((T, 2, NP, HD), jnp.float32),
    )(x, deg3, w)


def _tc_mid(g, yt, deg3, b, w):
    return pl.pallas_call(
        _tc_mid_body,
        grid=(T, NB),
        in_specs=[
            pl.BlockSpec((1, 2, RB, HD), lambda t, n: (t, 0, n, 0)),
            pl.BlockSpec((1, 2, RB, HD), lambda t, n: (t, 0, n, 0)),
            pl.BlockSpec((1, RB, 1), lambda t, n: (t, n, 0)),
            pl.BlockSpec((D,), lambda t, n: (0,)),
            pl.BlockSpec((D, D), lambda t, n: (0, 0)),
        ],
        out_specs=pl.BlockSpec((1, 2, RB, HD), lambda t, n: (t, 0, n, 0)),
        out_shape=jax.ShapeDtypeStruct((T, 2, NP, HD), jnp.float32),
    )(g, yt, deg3, b, w)


def _tc_final(g, yt, deg3, b):
    return pl.pallas_call(
        _tc_final_body,
        grid=(T, NBF),
        in_specs=[
            pl.BlockSpec((1, 2, RBF, HD), lambda t, n: (t, 0, n, 0)),
            pl.BlockSpec((1, 2, RBF, HD), lambda t, n: (t, 0, n, 0)),
            pl.BlockSpec((1, RBF, 1), lambda t, n: (t, n, 0)),
            pl.BlockSpec((D,), lambda t, n: (0,)),
        ],
        out_specs=pl.BlockSpec((1, RBF, D), lambda t, n: (t, n, 0)),
        out_shape=jax.ShapeDtypeStruct((T, N, D), jnp.float32),
    )(g, yt, deg3, b)


def kernel(node_features, edge_indices, edge_weights, emb_table,
           W1, b1, W2, b2, W3, b3):
    nf = jnp.pad(node_features, ((0, 0), (0, NP - N)))
    nf_flat = nf.reshape(T * NP)
    src = jnp.pad(edge_indices[:, 0, :], ((0, 0), (0, EP - E)))
    dst = jnp.pad(edge_indices[:, 1, :], ((0, 0), (0, EP - E)))
    ew = jnp.pad(edge_weights, ((0, 0), (0, EP - E)))
    src3 = src.reshape(T, EP // CH, CH)
    dst3 = dst.reshape(T, EP // CH, CH)
    ew3 = ew.reshape(T, EP // CH, CH)

    x, deg = _sc_prep(emb_table, nf_flat, dst3, ew3)
    deg3 = deg.reshape(T, NP, 1)

    yt1 = _tc1(x, deg3, W1)
    g1 = _sc_msg(yt1, src3, dst3, ew3)
    yt2 = _tc_mid(g1, yt1, deg3, b1, W2)
    g2 = _sc_msg(yt2, src3, dst3, ew3)
    yt3 = _tc_mid(g2, yt2, deg3, b2, W3)
    g3 = _sc_msg(yt3, src3, dst3, ew3)
    out = _tc_final(g3, yt3, deg3, b3)
    return out.reshape(T * N, D)
